# baseline (speedup 1.0000x reference)
import jax
import jax.numpy as jnp
from jax import lax
from jax.experimental import pallas as pl
from jax.experimental.pallas import tpu as pltpu
from jax.experimental.pallas import tpu_sc as plsc

NUM_NODES = 1000000
MEMORY_DIM = 64
BATCH = 16384

_NC = 2
_NS = 16
_NW = _NC * _NS
_BPW = BATCH // _NW


_mesh = plsc.VectorSubcoreMesh(core_axis_name="c", subcore_axis_name="s")


@pl.kernel(
    out_type=(
        jax.ShapeDtypeStruct((BATCH, MEMORY_DIM), jnp.float32),
        jax.ShapeDtypeStruct((BATCH,), jnp.int32),
    ),
    mesh=_mesh,
    compiler_params=pltpu.CompilerParams(use_tc_tiling_on_sc=False),
    scratch_types=[
        pltpu.VMEM((_BPW,), jnp.int32),
        pltpu.VMEM((_BPW, MEMORY_DIM), jnp.float32),
        pltpu.VMEM((_BPW,), jnp.int32),
        pltpu.SemaphoreType.DMA,
        pltpu.SemaphoreType.DMA,
    ],
)
def _gather_kernel(n_id_hbm, memory_hbm, last_hbm, mem_out_hbm, last_out_hbm,
                   idx_v, rows_v, last_v, sem_rows, sem_last):
    wid = lax.axis_index("s") * _NC + lax.axis_index("c")
    base = wid * _BPW
    pltpu.sync_copy(n_id_hbm.at[pl.ds(base, _BPW)], idx_v)
    cp_rows = pltpu.async_copy(memory_hbm.at[idx_v], rows_v, sem_rows)
    cp_last = pltpu.async_copy(last_hbm.at[idx_v], last_v, sem_last)
    cp_last.wait()
    pltpu.sync_copy(last_v, last_out_hbm.at[pl.ds(base, _BPW)])
    cp_rows.wait()
    pltpu.sync_copy(rows_v, mem_out_hbm.at[pl.ds(base, _BPW)])


def kernel(n_id, memory, last_update):
    mem_out, last_out = _gather_kernel(n_id, memory, last_update)
    return (mem_out, last_out, jnp.array(0, dtype=jnp.int32))
